# baseline (speedup 1.0000x reference)
import jax
import jax.numpy as jnp
from jax import lax
from jax.experimental import pallas as pl
from jax.experimental.pallas import tpu as pltpu
from jax.experimental.pallas import tpu_sc as plsc

NMS_T = 0.5
SCORE_T = 0.05
DETS = 100
NEG = -1e9
BIGIDX = float(2 ** 30)
LANES = 16
UNROLL = 4

_axis_index = lax.axis_index


def _nms_body(x1h, y1h, x2h, y2h, sh, outh,
              x1v, y1v, x2v, y2v, sv, av, outv, selflag):
    chunk = x1v.shape[0]
    vecs = chunk // LANES
    c = _axis_index("c")
    t = _axis_index("s")

    @pl.when((c == 0) & (t == 0))
    def _work():
        pltpu.sync_copy(x1h, x1v)
        pltpu.sync_copy(y1h, y1v)
        pltpu.sync_copy(x2h, x2v)
        pltpu.sync_copy(y2h, y2v)
        pltpu.sync_copy(sh, sv)

        def _areas(j, carry):
            sl = pl.ds(j * LANES, LANES)
            av[sl] = (x2v[sl] - x1v[sl]) * (y2v[sl] - y1v[sl])
            return carry
        lax.fori_loop(0, vecs, _areas, 0)

        def _z(j, carry):
            selflag[pl.ds(j * LANES, LANES)] = jnp.zeros((LANES,), jnp.int32)
            return carry
        lax.fori_loop(0, 256 // LANES, _z, 0)

        lanes = lax.iota(jnp.int32, LANES)
        ninf = jnp.full((LANES,), -jnp.inf, jnp.float32)
        bigv = jnp.full((LANES,), BIGIDX, jnp.float32)

        def _track(sj, gif, acc):
            accv, acci = acc
            take = sj > accv
            return jnp.where(take, sj, accv), jnp.where(take, gif, acci)

        def _reduce_winner(accv, acci):
            mval = jnp.max(accv)
            gidxf = jnp.min(jnp.where(accv == mval, acci, bigv))
            lidx = gidxf.astype(jnp.int32)
            lsplat = jnp.full((LANES,), lidx, jnp.int32)
            bx1 = plsc.load_gather(x1v, [lsplat])
            by1 = plsc.load_gather(y1v, [lsplat])
            bx2 = plsc.load_gather(x2v, [lsplat])
            by2 = plsc.load_gather(y2v, [lsplat])
            ba = plsc.load_gather(av, [lsplat])
            rvec = jnp.where(lanes == 0, bx1,
                   jnp.where(lanes == 1, by1,
                   jnp.where(lanes == 2, bx2,
                   jnp.where(lanes == 3, by2,
                   jnp.where(lanes == 4, mval,
                   jnp.where(lanes == 5, gidxf, ba))))))
            return rvec

        def _emit(m, rvec):
            wval = rvec[4]
            valid = wval > SCORE_T

            @pl.when(valid)
            def _():
                outv[pl.ds(m * LANES, LANES)] = rvec
                plsc.store_scatter(
                    selflag,
                    [jnp.full((LANES,), rvec[5].astype(jnp.int32), jnp.int32)],
                    jnp.ones((LANES,), jnp.int32), mask=lanes == 0)
            return m + jnp.where(valid, jnp.int32(1), jnp.int32(0))

        @plsc.parallel_loop(0, vecs, unroll=UNROLL, carry=(ninf, bigv))
        def _acc0(jj, acc):
            sl = pl.ds(jj * LANES, LANES)
            gif = (jj * LANES + lanes).astype(jnp.float32)
            return _track(sv[sl], gif, acc)
        accv, acci = _acc0
        rvec = _reduce_winner(accv, acci)
        m0 = _emit(jnp.int32(0), rvec)

        def _iter(k, carry):
            m, wrec = carry
            wx1 = wrec[0]
            wy1 = wrec[1]
            wx2 = wrec[2]
            wy2 = wrec[3]
            wa = wrec[6]

            @plsc.parallel_loop(0, vecs, unroll=UNROLL, carry=(ninf, bigv))
            def _accs(jj, acc):
                sl = pl.ds(jj * LANES, LANES)
                xx1 = jnp.maximum(wx1, x1v[sl])
                yy1 = jnp.maximum(wy1, y1v[sl])
                xx2 = jnp.minimum(wx2, x2v[sl])
                yy2 = jnp.minimum(wy2, y2v[sl])
                inter = (jnp.maximum(xx2 - xx1, 0.0) *
                         jnp.maximum(yy2 - yy1, 0.0))
                iou = inter / (wa + av[sl] - inter + 1e-9)
                snew = jnp.where(iou > NMS_T, NEG, sv[sl])
                sv[sl] = snew
                gif = (jj * LANES + lanes).astype(jnp.float32)
                return _track(snew, gif, acc)
            accv, acci = _accs
            rvec = _reduce_winner(accv, acci)
            return _emit(m, rvec), rvec

        m, _ = lax.fori_loop(1, DETS, _iter, (m0, rvec))

        def _fb(j, run):
            sl = pl.ds(j * LANES, LANES)
            z = selflag[sl] == 0
            inc = z.astype(jnp.int32)
            cum = plsc.cumsum(inc)
            slot = m + run + (cum - inc)
            en = z & (slot < DETS)
            sbase = slot * LANES
            plsc.store_scatter(outv, [sbase + 0], x1v[sl], mask=en)
            plsc.store_scatter(outv, [sbase + 1], y1v[sl], mask=en)
            plsc.store_scatter(outv, [sbase + 2], x2v[sl], mask=en)
            plsc.store_scatter(outv, [sbase + 3], y2v[sl], mask=en)
            plsc.store_scatter(outv, [sbase + 4],
                               jnp.full((LANES,), NEG, jnp.float32),
                               mask=en)
            return run + cum[LANES - 1]
        lax.fori_loop(0, 256 // LANES, _fb, jnp.int32(0))
        pltpu.sync_copy(outv, outh)


def _build(n, interpret=False):
    npad = -(-n // (LANES * UNROLL)) * LANES * UNROLL
    mesh = plsc.VectorSubcoreMesh(
        core_axis_name="c", subcore_axis_name="s", num_cores=1)
    f = pl.kernel(
        _nms_body,
        out_type=jax.ShapeDtypeStruct((DETS * LANES,), jnp.float32),
        mesh=mesh,
        compiler_params=pltpu.CompilerParams(needs_layout_passes=False),
        interpret=interpret,
        scratch_types=[
            pltpu.VMEM((npad,), jnp.float32),
            pltpu.VMEM((npad,), jnp.float32),
            pltpu.VMEM((npad,), jnp.float32),
            pltpu.VMEM((npad,), jnp.float32),
            pltpu.VMEM((npad,), jnp.float32),
            pltpu.VMEM((npad,), jnp.float32),
            pltpu.VMEM((DETS * LANES,), jnp.float32),
            pltpu.VMEM((npad,), jnp.int32),
        ],
    )
    return f, npad


def kernel(boxes, scores):
    n = boxes.shape[0]
    f, npad = _build(n)
    x1 = jnp.pad(boxes[:, 0], (0, npad - n))
    y1 = jnp.pad(boxes[:, 1], (0, npad - n))
    x2 = jnp.pad(boxes[:, 2], (0, npad - n))
    y2 = jnp.pad(boxes[:, 3], (0, npad - n))
    sp = jnp.pad(scores, (0, npad - n), constant_values=NEG)
    out = f(x1, y1, x2, y2, sp)
    return out.reshape(DETS, LANES)[:, :5]


# speedup vs baseline: 2.1452x; 2.1452x over previous
import jax
import jax.numpy as jnp
from jax import lax
from jax.experimental import pallas as pl
from jax.experimental.pallas import tpu as pltpu
from jax.experimental.pallas import tpu_sc as plsc

NMS_T = 0.5
SCORE_T = 0.05
DETS = 100
NEG = -1e9
BIGIDX = float(2 ** 30)
LANES = 16
UNROLL = 4
NTILES = 16
RECW = 128

_axis_index = lax.axis_index


def _nms_body(x1h, y1h, x2h, y2h, sh, outh,
              x1v, y1v, x2v, y2v, sv, av, rec, allrec, outv, selflag, cand):
    chunk = x1v.shape[0]
    vecs = chunk // LANES
    c = _axis_index("c")
    t = _axis_index("s")
    multi = NTILES > 1
    active = (c == 0) if multi else (c == 0) & (t == 0)

    @pl.when(active)
    def _work():
        base = t * chunk
        pltpu.sync_copy(x1h.at[pl.ds(base, chunk)], x1v)
        pltpu.sync_copy(y1h.at[pl.ds(base, chunk)], y1v)
        pltpu.sync_copy(x2h.at[pl.ds(base, chunk)], x2v)
        pltpu.sync_copy(y2h.at[pl.ds(base, chunk)], y2v)
        pltpu.sync_copy(sh.at[pl.ds(base, chunk)], sv)

        lanes = lax.iota(jnp.int32, LANES)
        ninf = jnp.full((LANES,), -jnp.inf, jnp.float32)
        bigv = jnp.full((LANES,), BIGIDX, jnp.float32)

        def _sweep(fn, acc):
            if vecs <= 24:
                for jj in range(vecs):
                    acc = fn(jj, acc)
                return acc

            @plsc.parallel_loop(0, vecs, unroll=UNROLL, carry=acc)
            def _r(jj, a):
                return fn(jj, a)
            return _r

        def _area(jj, acc):
            sl = pl.ds(jj * LANES, LANES)
            av[sl] = (x2v[sl] - x1v[sl]) * (y2v[sl] - y1v[sl])
            return acc
        _sweep(_area, jnp.int32(0))

        @pl.when(t == 0)
        def _zero():
            for jj in range(256 // LANES):
                selflag[pl.ds(jj * LANES, LANES)] = jnp.zeros(
                    (LANES,), jnp.int32)

        def _track(sj, gif, acc):
            accv, acci = acc
            take = sj > accv
            return jnp.where(take, sj, accv), jnp.where(take, gif, acci)

        def _local_record(acc):
            accv, acci = acc
            mval = jnp.max(accv)
            gidxf = jnp.min(jnp.where(accv == mval, acci, bigv))
            lidx = gidxf.astype(jnp.int32) - base
            lsplat = jnp.full((LANES,), lidx, jnp.int32)
            bx1 = plsc.load_gather(x1v, [lsplat])
            by1 = plsc.load_gather(y1v, [lsplat])
            bx2 = plsc.load_gather(x2v, [lsplat])
            by2 = plsc.load_gather(y2v, [lsplat])
            ba = plsc.load_gather(av, [lsplat])
            return jnp.where(lanes == 0, bx1,
                   jnp.where(lanes == 1, by1,
                   jnp.where(lanes == 2, bx2,
                   jnp.where(lanes == 3, by2,
                   jnp.where(lanes == 4, mval,
                   jnp.where(lanes == 5, gidxf, ba))))))

        def _exchange(rvec):
            if not multi:
                return rvec
            rec[pl.ds(0, LANES)] = rvec
            pltpu.sync_copy(rec, cand.at[t])
            plsc.subcore_barrier()
            pltpu.sync_copy(cand, allrec)
            plsc.subcore_barrier()
            svec = plsc.load_gather(
                allrec, [lanes, jnp.full((LANES,), 4, jnp.int32)])
            ivec = plsc.load_gather(
                allrec, [lanes, jnp.full((LANES,), 5, jnp.int32)])
            wval = jnp.max(svec)
            wsel = svec == wval
            widxf = jnp.min(jnp.where(wsel, ivec, bigv))
            wmask = wsel & (ivec == widxf)
            wtile = jnp.min(jnp.where(wmask, lanes, jnp.int32(LANES - 1)))
            return plsc.load_gather(
                allrec, [jnp.full((LANES,), wtile, jnp.int32), lanes])

        def _emit(m, wrec):
            wval = wrec[4]
            valid = wval > SCORE_T

            @pl.when((t == 0) & valid)
            def _():
                outv[pl.ds(m * LANES, LANES)] = wrec
                plsc.store_scatter(
                    selflag,
                    [jnp.full((LANES,), wrec[5].astype(jnp.int32), jnp.int32)],
                    jnp.ones((LANES,), jnp.int32), mask=lanes == 0)
            return m + jnp.where(valid, jnp.int32(1), jnp.int32(0))

        def _mx0(jj, acc):
            sl = pl.ds(jj * LANES, LANES)
            gif = (base + jj * LANES + lanes).astype(jnp.float32)
            return _track(sv[sl], gif, acc)
        wrec0 = _exchange(_local_record(_sweep(_mx0, (ninf, bigv))))
        m0 = _emit(jnp.int32(0), wrec0)

        def _iter(k, carry):
            m, wrec = carry
            wx1 = wrec[0]
            wy1 = wrec[1]
            wx2 = wrec[2]
            wy2 = wrec[3]
            wa = wrec[6]

            def _fp(jj, acc):
                sl = pl.ds(jj * LANES, LANES)
                xx1 = jnp.maximum(wx1, x1v[sl])
                yy1 = jnp.maximum(wy1, y1v[sl])
                xx2 = jnp.minimum(wx2, x2v[sl])
                yy2 = jnp.minimum(wy2, y2v[sl])
                inter = (jnp.maximum(xx2 - xx1, 0.0) *
                         jnp.maximum(yy2 - yy1, 0.0))
                iou = inter / (wa + av[sl] - inter + 1e-9)
                snew = jnp.where(iou > NMS_T, NEG, sv[sl])
                sv[sl] = snew
                gif = (base + jj * LANES + lanes).astype(jnp.float32)
                return _track(snew, gif, acc)
            nwrec = _exchange(_local_record(_sweep(_fp, (ninf, bigv))))
            return _emit(m, nwrec), nwrec

        m, _ = lax.fori_loop(1, DETS, _iter, (m0, wrec0))

        @pl.when(t == 0)
        def _finish():
            def _fb(jj, run):
                sl = pl.ds(jj * LANES, LANES)
                z = selflag[sl] == 0
                inc = z.astype(jnp.int32)
                cum = plsc.cumsum(inc)
                slot = m + run + (cum - inc)
                en = z & (slot < DETS)
                sbase = slot * LANES
                plsc.store_scatter(outv, [sbase + 0], x1v[sl], mask=en)
                plsc.store_scatter(outv, [sbase + 1], y1v[sl], mask=en)
                plsc.store_scatter(outv, [sbase + 2], x2v[sl], mask=en)
                plsc.store_scatter(outv, [sbase + 3], y2v[sl], mask=en)
                plsc.store_scatter(outv, [sbase + 4],
                                   jnp.full((LANES,), NEG, jnp.float32),
                                   mask=en)
                return run + cum[LANES - 1]
            lax.fori_loop(0, 256 // LANES, _fb, jnp.int32(0))
            pltpu.sync_copy(outv, outh)


def _build(n, interpret=False):
    grain = LANES * UNROLL * NTILES
    npad = -(-n // grain) * grain
    chunk = npad // NTILES
    mesh = plsc.VectorSubcoreMesh(
        core_axis_name="c", subcore_axis_name="s", num_cores=1)
    f = pl.kernel(
        _nms_body,
        out_type=jax.ShapeDtypeStruct((DETS * LANES,), jnp.float32),
        mesh=mesh,
        compiler_params=pltpu.CompilerParams(needs_layout_passes=False),
        interpret=interpret,
        scratch_types=[
            pltpu.VMEM((chunk,), jnp.float32),
            pltpu.VMEM((chunk,), jnp.float32),
            pltpu.VMEM((chunk,), jnp.float32),
            pltpu.VMEM((chunk,), jnp.float32),
            pltpu.VMEM((chunk,), jnp.float32),
            pltpu.VMEM((chunk,), jnp.float32),
            pltpu.VMEM((RECW,), jnp.float32),
            pltpu.VMEM((NTILES, RECW), jnp.float32),
            pltpu.VMEM((DETS * LANES,), jnp.float32),
            pltpu.VMEM((npad,), jnp.int32),
            pltpu.VMEM_SHARED((NTILES, RECW), jnp.float32),
        ],
    )
    return f, npad


def kernel(boxes, scores):
    n = boxes.shape[0]
    f, npad = _build(n)
    x1 = jnp.pad(boxes[:, 0], (0, npad - n))
    y1 = jnp.pad(boxes[:, 1], (0, npad - n))
    x2 = jnp.pad(boxes[:, 2], (0, npad - n))
    y2 = jnp.pad(boxes[:, 3], (0, npad - n))
    sp = jnp.pad(scores, (0, npad - n), constant_values=NEG)
    out = f(x1, y1, x2, y2, sp)
    return out.reshape(DETS, LANES)[:, :5]


# speedup vs baseline: 2.2583x; 1.0527x over previous
import jax
import jax.numpy as jnp
from jax import lax
from jax.experimental import pallas as pl
from jax.experimental.pallas import tpu as pltpu
from jax.experimental.pallas import tpu_sc as plsc

NMS_T = 0.5
SCORE_T = 0.05
DETS = 100
NEG = -1e9
BIGIDX = float(2 ** 30)
LANES = 16
UNROLL = 4
NTILES = 16
RECW = 128

_axis_index = lax.axis_index


def _nms_body(x1h, y1h, x2h, y2h, sh, outh,
              x1v, y1v, x2v, y2v, sv, av, rec, allrec, outv, selflag, cand):
    chunk = x1v.shape[0]
    vecs = chunk // LANES
    c = _axis_index("c")
    t = _axis_index("s")
    multi = NTILES > 1
    active = (c == 0) if multi else (c == 0) & (t == 0)

    @pl.when(active)
    def _work():
        base = t * chunk
        pltpu.sync_copy(x1h.at[pl.ds(base, chunk)], x1v)
        pltpu.sync_copy(y1h.at[pl.ds(base, chunk)], y1v)
        pltpu.sync_copy(x2h.at[pl.ds(base, chunk)], x2v)
        pltpu.sync_copy(y2h.at[pl.ds(base, chunk)], y2v)
        pltpu.sync_copy(sh.at[pl.ds(base, chunk)], sv)

        lanes = lax.iota(jnp.int32, LANES)
        ninf = jnp.full((LANES,), -jnp.inf, jnp.float32)
        bigv = jnp.full((LANES,), BIGIDX, jnp.float32)

        def _sweep(fn, acc):
            if vecs <= 24:
                for jj in range(vecs):
                    acc = fn(jj, acc)
                return acc

            @plsc.parallel_loop(0, vecs, unroll=UNROLL, carry=acc)
            def _r(jj, a):
                return fn(jj, a)
            return _r

        def _area(jj, acc):
            sl = pl.ds(jj * LANES, LANES)
            av[sl] = (x2v[sl] - x1v[sl]) * (y2v[sl] - y1v[sl])
            return acc
        _sweep(_area, jnp.int32(0))

        @pl.when(t == 0)
        def _zero():
            for jj in range(256 // LANES):
                selflag[pl.ds(jj * LANES, LANES)] = jnp.zeros(
                    (LANES,), jnp.int32)

        def _track(sj, gif, acc):
            accv, acci = acc
            take = sj > accv
            return jnp.where(take, sj, accv), jnp.where(take, gif, acci)

        def _local_record(acc):
            accv, acci = acc
            mval = jnp.max(accv)
            gidxf = jnp.min(jnp.where(accv == mval, acci, bigv))
            lidx = gidxf.astype(jnp.int32) - base
            lsplat = jnp.full((LANES,), lidx, jnp.int32)
            bx1 = plsc.load_gather(x1v, [lsplat])
            by1 = plsc.load_gather(y1v, [lsplat])
            bx2 = plsc.load_gather(x2v, [lsplat])
            by2 = plsc.load_gather(y2v, [lsplat])
            ba = plsc.load_gather(av, [lsplat])
            return jnp.where(lanes == 0, bx1,
                   jnp.where(lanes == 1, by1,
                   jnp.where(lanes == 2, bx2,
                   jnp.where(lanes == 3, by2,
                   jnp.where(lanes == 4, mval,
                   jnp.where(lanes == 5, gidxf, ba))))))

        def _exchange(rvec, pb):
            if not multi:
                return rvec
            rec[pl.ds(0, LANES)] = rvec
            pltpu.sync_copy(rec, cand.at[pb, t])
            plsc.subcore_barrier()
            pltpu.sync_copy(cand.at[pb], allrec)
            svec = plsc.load_gather(
                allrec, [lanes, jnp.full((LANES,), 4, jnp.int32)])
            ivec = plsc.load_gather(
                allrec, [lanes, jnp.full((LANES,), 5, jnp.int32)])
            wval = jnp.max(svec)
            wsel = svec == wval
            widxf = jnp.min(jnp.where(wsel, ivec, bigv))
            wmask = wsel & (ivec == widxf)
            wtile = jnp.min(jnp.where(wmask, lanes, jnp.int32(LANES - 1)))
            return plsc.load_gather(
                allrec, [jnp.full((LANES,), wtile, jnp.int32), lanes])

        def _emit(m, wrec):
            wval = wrec[4]
            valid = wval > SCORE_T

            @pl.when((t == 0) & valid)
            def _():
                outv[pl.ds(m * LANES, LANES)] = wrec
                plsc.store_scatter(
                    selflag,
                    [jnp.full((LANES,), wrec[5].astype(jnp.int32), jnp.int32)],
                    jnp.ones((LANES,), jnp.int32), mask=lanes == 0)
            return m + jnp.where(valid, jnp.int32(1), jnp.int32(0))

        def _mx0(jj, acc):
            sl = pl.ds(jj * LANES, LANES)
            gif = (base + jj * LANES + lanes).astype(jnp.float32)
            return _track(sv[sl], gif, acc)
        wrec0 = _exchange(_local_record(_sweep(_mx0, (ninf, bigv))),
                          jnp.int32(0))
        m0 = _emit(jnp.int32(0), wrec0)

        def _iter(k, carry):
            m, wrec = carry
            wx1 = wrec[0]
            wy1 = wrec[1]
            wx2 = wrec[2]
            wy2 = wrec[3]
            wa = wrec[6]

            def _fp(jj, acc):
                sl = pl.ds(jj * LANES, LANES)
                xx1 = jnp.maximum(wx1, x1v[sl])
                yy1 = jnp.maximum(wy1, y1v[sl])
                xx2 = jnp.minimum(wx2, x2v[sl])
                yy2 = jnp.minimum(wy2, y2v[sl])
                inter = (jnp.maximum(xx2 - xx1, 0.0) *
                         jnp.maximum(yy2 - yy1, 0.0))
                iou = inter / (wa + av[sl] - inter + 1e-9)
                snew = jnp.where(iou > NMS_T, NEG, sv[sl])
                sv[sl] = snew
                gif = (base + jj * LANES + lanes).astype(jnp.float32)
                return _track(snew, gif, acc)
            nwrec = _exchange(_local_record(_sweep(_fp, (ninf, bigv))),
                              k & 1)
            return _emit(m, nwrec), nwrec

        m, _ = lax.fori_loop(1, DETS, _iter, (m0, wrec0))

        @pl.when(t == 0)
        def _finish():
            def _fb(jj, run):
                sl = pl.ds(jj * LANES, LANES)
                z = selflag[sl] == 0
                inc = z.astype(jnp.int32)
                cum = plsc.cumsum(inc)
                slot = m + run + (cum - inc)
                en = z & (slot < DETS)
                sbase = slot * LANES
                plsc.store_scatter(outv, [sbase + 0], x1v[sl], mask=en)
                plsc.store_scatter(outv, [sbase + 1], y1v[sl], mask=en)
                plsc.store_scatter(outv, [sbase + 2], x2v[sl], mask=en)
                plsc.store_scatter(outv, [sbase + 3], y2v[sl], mask=en)
                plsc.store_scatter(outv, [sbase + 4],
                                   jnp.full((LANES,), NEG, jnp.float32),
                                   mask=en)
                return run + cum[LANES - 1]
            lax.fori_loop(0, 256 // LANES, _fb, jnp.int32(0))
            pltpu.sync_copy(outv, outh)


def _build(n, interpret=False):
    grain = LANES * UNROLL * NTILES
    npad = -(-n // grain) * grain
    chunk = npad // NTILES
    mesh = plsc.VectorSubcoreMesh(
        core_axis_name="c", subcore_axis_name="s", num_cores=1)
    f = pl.kernel(
        _nms_body,
        out_type=jax.ShapeDtypeStruct((DETS * LANES,), jnp.float32),
        mesh=mesh,
        compiler_params=pltpu.CompilerParams(needs_layout_passes=False),
        interpret=interpret,
        scratch_types=[
            pltpu.VMEM((chunk,), jnp.float32),
            pltpu.VMEM((chunk,), jnp.float32),
            pltpu.VMEM((chunk,), jnp.float32),
            pltpu.VMEM((chunk,), jnp.float32),
            pltpu.VMEM((chunk,), jnp.float32),
            pltpu.VMEM((chunk,), jnp.float32),
            pltpu.VMEM((RECW,), jnp.float32),
            pltpu.VMEM((NTILES, RECW), jnp.float32),
            pltpu.VMEM((DETS * LANES,), jnp.float32),
            pltpu.VMEM((npad,), jnp.int32),
            pltpu.VMEM_SHARED((2, NTILES, RECW), jnp.float32),
        ],
    )
    return f, npad


def kernel(boxes, scores):
    n = boxes.shape[0]
    f, npad = _build(n)
    x1 = jnp.pad(boxes[:, 0], (0, npad - n))
    y1 = jnp.pad(boxes[:, 1], (0, npad - n))
    x2 = jnp.pad(boxes[:, 2], (0, npad - n))
    y2 = jnp.pad(boxes[:, 3], (0, npad - n))
    sp = jnp.pad(scores, (0, npad - n), constant_values=NEG)
    out = f(x1, y1, x2, y2, sp)
    return out.reshape(DETS, LANES)[:, :5]


# speedup vs baseline: 2.5294x; 1.1200x over previous
import jax
import jax.numpy as jnp
from jax import lax
from jax.experimental import pallas as pl
from jax.experimental.pallas import tpu as pltpu
from jax.experimental.pallas import tpu_sc as plsc

NMS_T = 0.5
SCORE_T = 0.05
DETS = 100
NEG = -1e9
BIGIDX = float(2 ** 30)
LANES = 16
NTILES = 16
RECW = 128

_axis_index = lax.axis_index


def _nms_body(x1h, y1h, x2h, y2h, sh, outh,
              x1v, y1v, x2v, y2v, sv, av, rec, allrec, outv, selflag, cand):
    chunk = x1v.shape[0]
    vecs = chunk // LANES
    c = _axis_index("c")
    t = _axis_index("s")

    @pl.when(c == 0)
    def _work():
        base = t * chunk
        pltpu.sync_copy(x1h.at[pl.ds(base, chunk)], x1v)
        pltpu.sync_copy(y1h.at[pl.ds(base, chunk)], y1v)
        pltpu.sync_copy(x2h.at[pl.ds(base, chunk)], x2v)
        pltpu.sync_copy(y2h.at[pl.ds(base, chunk)], y2v)
        pltpu.sync_copy(sh.at[pl.ds(base, chunk)], sv)

        lanes = lax.iota(jnp.int32, LANES)
        ninf = jnp.full((LANES,), -jnp.inf, jnp.float32)
        bigv = jnp.full((LANES,), BIGIDX, jnp.float32)

        for jj in range(vecs):
            sl = pl.ds(jj * LANES, LANES)
            av[sl] = (x2v[sl] - x1v[sl]) * (y2v[sl] - y1v[sl])

        @pl.when(t == 0)
        def _zero():
            for jj in range(256 // LANES):
                selflag[pl.ds(jj * LANES, LANES)] = jnp.zeros(
                    (LANES,), jnp.int32)

        def _iou_vec(wx1, wy1, wx2, wy2, wa, bx1, by1, bx2, by2, ba):
            xx1 = jnp.maximum(wx1, bx1)
            yy1 = jnp.maximum(wy1, by1)
            xx2 = jnp.minimum(wx2, bx2)
            yy2 = jnp.minimum(wy2, by2)
            inter = (jnp.maximum(xx2 - xx1, 0.0) *
                     jnp.maximum(yy2 - yy1, 0.0))
            return inter / (wa + ba - inter + 1e-9)

        def _round(carry):
            r, m, w1rec, w2rec = carry
            ax1, ay1, ax2, ay2, aa = (w1rec[0], w1rec[1], w1rec[2],
                                      w1rec[3], w1rec[6])
            bx1w, by1w, bx2w, by2w, ba_w = (w2rec[0], w2rec[1], w2rec[2],
                                            w2rec[3], w2rec[6])

            a1v, a1i, a2v, a2i = ninf, bigv, ninf, bigv
            for jj in range(vecs):
                sl = pl.ds(jj * LANES, LANES)
                px1, py1 = x1v[sl], y1v[sl]
                px2, py2 = x2v[sl], y2v[sl]
                pa = av[sl]
                qa = _iou_vec(ax1, ay1, ax2, ay2, aa, px1, py1, px2, py2, pa)
                qb = _iou_vec(bx1w, by1w, bx2w, by2w, ba_w,
                              px1, py1, px2, py2, pa)
                snew = jnp.where((qa > NMS_T) | (qb > NMS_T), NEG, sv[sl])
                sv[sl] = snew
                gif = (base + jj * LANES + lanes).astype(jnp.float32)
                tk1 = snew > a1v
                tk2 = jnp.logical_not(tk1) & (snew > a2v)
                a2v = jnp.where(tk1, a1v, jnp.where(tk2, snew, a2v))
                a2i = jnp.where(tk1, a1i, jnp.where(tk2, gif, a2i))
                a1v = jnp.where(tk1, snew, a1v)
                a1i = jnp.where(tk1, gif, a1i)

            m1 = jnp.max(a1v)
            i1 = jnp.min(jnp.where(a1v == m1, a1i, bigv))
            l1mask = (a1v == m1) & (a1i == i1)
            u2v = jnp.where(l1mask, a2v, a1v)
            u2i = jnp.where(l1mask, a2i, a1i)
            m2 = jnp.max(u2v)
            i2 = jnp.min(jnp.where(u2v == m2, u2i, bigv))

            lidx1 = i1.astype(jnp.int32) - base
            lidx2 = i2.astype(jnp.int32) - base
            lsplat = jnp.where(lanes < 8, lidx1, lidx2)
            gx1 = plsc.load_gather(x1v, [lsplat])
            gy1 = plsc.load_gather(y1v, [lsplat])
            gx2 = plsc.load_gather(x2v, [lsplat])
            gy2 = plsc.load_gather(y2v, [lsplat])
            ga = plsc.load_gather(av, [lsplat])
            l8 = lanes & 7
            svals = jnp.where(lanes < 8, m1, m2)
            ivals = jnp.where(lanes < 8, i1, i2)
            rvec = jnp.where(l8 == 0, gx1,
                   jnp.where(l8 == 1, gy1,
                   jnp.where(l8 == 2, gx2,
                   jnp.where(l8 == 3, gy2,
                   jnp.where(l8 == 4, svals,
                   jnp.where(l8 == 5, ivals, ga))))))

            pb = r & 1
            rec[pl.ds(0, LANES)] = rvec
            pltpu.sync_copy(rec, cand.at[pb, t])
            plsc.subcore_barrier()
            pltpu.sync_copy(cand.at[pb], allrec)

            def col(j):
                return plsc.load_gather(
                    allrec, [lanes, jnp.full((LANES,), j, jnp.int32)])

            s1c, i1c = col(4), col(5)
            wv1 = jnp.max(s1c)
            ws1 = s1c == wv1
            wi1 = jnp.min(jnp.where(ws1, i1c, bigv))
            wm1 = ws1 & (i1c == wi1)
            wt1 = jnp.min(jnp.where(wm1, lanes, jnp.int32(LANES - 1)))
            nw1 = plsc.load_gather(
                allrec, [jnp.full((LANES,), wt1, jnp.int32), lanes])

            nx1, ny1, nx2, ny2, na = (nw1[0], nw1[1], nw1[2], nw1[3], nw1[6])
            q1 = _iou_vec(nx1, ny1, nx2, ny2, na,
                          col(0), col(1), col(2), col(3), col(6))
            q2 = _iou_vec(nx1, ny1, nx2, ny2, na,
                          col(8), col(9), col(10), col(11), col(14))
            surv1 = jnp.logical_not(q1 > NMS_T)
            surv2 = jnp.logical_not(q2 > NMS_T)
            is1 = lanes == wt1
            s2c, i2c = col(12), col(13)
            bestv = jnp.where(is1,
                              jnp.where(surv2, s2c, ninf),
                              jnp.where(surv1, s1c,
                                        jnp.where(surv2, s2c, ninf)))
            besti = jnp.where(is1, i2c, jnp.where(surv1, i1c, i2c))
            use2 = is1 | jnp.logical_not(surv1)
            unk = jnp.where(is1, jnp.logical_not(surv2),
                            jnp.logical_not(surv1 | surv2))
            sound = jnp.max(unk.astype(jnp.int32)) == 0

            wv2 = jnp.max(bestv)
            ws2 = bestv == wv2
            wi2 = jnp.min(jnp.where(ws2, besti, bigv))
            wm2 = ws2 & (besti == wi2)
            wt2 = jnp.min(jnp.where(wm2, lanes, jnp.int32(LANES - 1)))
            shift = jnp.max(jnp.where(wm2 & use2, jnp.int32(8), jnp.int32(0)))
            nw2 = plsc.load_gather(
                allrec, [jnp.full((LANES,), wt2, jnp.int32), lanes + shift])
            nw2 = jnp.where(sound, nw2, nw1)

            valid1 = nw1[4] > SCORE_T
            m1_ = m + jnp.where(valid1, jnp.int32(1), jnp.int32(0))
            valid2 = sound & (nw2[4] > SCORE_T) & (m1_ < DETS)
            m2_ = m1_ + jnp.where(valid2, jnp.int32(1), jnp.int32(0))

            @pl.when((t == 0) & valid1)
            def _():
                outv[pl.ds(m * LANES, LANES)] = nw1
                plsc.store_scatter(
                    selflag,
                    [jnp.full((LANES,), nw1[5].astype(jnp.int32), jnp.int32)],
                    jnp.ones((LANES,), jnp.int32), mask=lanes == 0)

            @pl.when((t == 0) & valid2)
            def _():
                outv[pl.ds(m1_ * LANES, LANES)] = nw2
                plsc.store_scatter(
                    selflag,
                    [jnp.full((LANES,), nw2[5].astype(jnp.int32), jnp.int32)],
                    jnp.ones((LANES,), jnp.int32), mask=lanes == 0)

            return r + 1, m2_, nw1, nw2

        zero_rec = jnp.zeros((LANES,), jnp.float32)

        def _cond(carry):
            r, m, _, _ = carry
            return (m < DETS) & (r < DETS)

        _, m, _, _ = lax.while_loop(
            _cond, _round, (jnp.int32(0), jnp.int32(0), zero_rec, zero_rec))

        @pl.when(t == 0)
        def _finish():
            def _fb(jj, run):
                sl = pl.ds(jj * LANES, LANES)
                z = selflag[sl] == 0
                inc = z.astype(jnp.int32)
                cum = plsc.cumsum(inc)
                slot = m + run + (cum - inc)
                en = z & (slot < DETS)
                sbase = slot * LANES
                plsc.store_scatter(outv, [sbase + 0], x1v[sl], mask=en)
                plsc.store_scatter(outv, [sbase + 1], y1v[sl], mask=en)
                plsc.store_scatter(outv, [sbase + 2], x2v[sl], mask=en)
                plsc.store_scatter(outv, [sbase + 3], y2v[sl], mask=en)
                plsc.store_scatter(outv, [sbase + 4],
                                   jnp.full((LANES,), NEG, jnp.float32),
                                   mask=en)
                return run + cum[LANES - 1]
            lax.fori_loop(0, 256 // LANES, _fb, jnp.int32(0))
            pltpu.sync_copy(outv, outh)


def _build(n, interpret=False):
    grain = LANES * NTILES
    npad = -(-n // grain) * grain
    chunk = npad // NTILES
    mesh = plsc.VectorSubcoreMesh(
        core_axis_name="c", subcore_axis_name="s", num_cores=1)
    f = pl.kernel(
        _nms_body,
        out_type=jax.ShapeDtypeStruct((DETS * LANES,), jnp.float32),
        mesh=mesh,
        compiler_params=pltpu.CompilerParams(needs_layout_passes=False),
        interpret=interpret,
        scratch_types=[
            pltpu.VMEM((chunk,), jnp.float32),
            pltpu.VMEM((chunk,), jnp.float32),
            pltpu.VMEM((chunk,), jnp.float32),
            pltpu.VMEM((chunk,), jnp.float32),
            pltpu.VMEM((chunk,), jnp.float32),
            pltpu.VMEM((chunk,), jnp.float32),
            pltpu.VMEM((RECW,), jnp.float32),
            pltpu.VMEM((NTILES, RECW), jnp.float32),
            pltpu.VMEM((DETS * LANES,), jnp.float32),
            pltpu.VMEM((npad,), jnp.int32),
            pltpu.VMEM_SHARED((2, NTILES, RECW), jnp.float32),
        ],
    )
    return f, npad


def kernel(boxes, scores):
    n = boxes.shape[0]
    f, npad = _build(n)
    x1 = jnp.pad(boxes[:, 0], (0, npad - n))
    y1 = jnp.pad(boxes[:, 1], (0, npad - n))
    x2 = jnp.pad(boxes[:, 2], (0, npad - n))
    y2 = jnp.pad(boxes[:, 3], (0, npad - n))
    sp = jnp.pad(scores, (0, npad - n), constant_values=NEG)
    out = f(x1, y1, x2, y2, sp)
    return out.reshape(DETS, LANES)[:, :5]
